# baseline (speedup 1.0000x reference)
import functools

import jax
import jax.numpy as jnp
from jax.experimental import pallas as pl


def _gelu_exact(v):
    return 0.5 * v * (1.0 + jax.lax.erf(v * jnp.float32(0.7071067811865476)))


def _body(nsteps, n_tokens, x_ref, w1_ref, b1_ref, w2_ref, b2_ref, wd_ref,
          rmat_ref, sel_ref, wup_ref, out_ref, aux_ref, psum_ref, lsum_ref):
    step = pl.program_id(0)
    xb = x_ref[0]
    E = rmat_ref.shape[0]

    xb_h = xb.astype(jnp.bfloat16)
    h = _gelu_exact(
        jnp.dot(w1_ref[...], xb_h, preferred_element_type=jnp.float32)
        + b1_ref[...])
    shared = (jnp.dot(w2_ref[...], h.astype(jnp.bfloat16),
                      preferred_element_type=jnp.float32)
              + b2_ref[...])

    feats = jnp.dot(wd_ref[...], xb, preferred_element_type=jnp.float32)
    logits = jnp.dot(rmat_ref[...], feats, preferred_element_type=jnp.float32)

    m = jnp.max(logits, axis=0, keepdims=True)
    ex = jnp.exp(logits - m)
    probs = ex / jnp.sum(ex, axis=0, keepdims=True)

    eidx = jax.lax.broadcasted_iota(jnp.int32, probs.shape, 0)
    m1 = jnp.max(probs, axis=0, keepdims=True)
    i1 = jnp.min(jnp.where(probs == m1, eidx, E), axis=0, keepdims=True)
    mask1 = eidx == i1
    rest = jnp.where(mask1, -1.0, probs)
    m2 = jnp.max(rest, axis=0, keepdims=True)
    i2 = jnp.min(jnp.where(rest == m2, eidx, E), axis=0, keepdims=True)
    sel = mask1 | (eidx == i2)
    gate = jnp.where(sel, probs, 0.0) / (m1 + m2)

    gate_big = jnp.dot(sel_ref[...], gate,
                       preferred_element_type=jnp.float32)
    wf = _gelu_exact(feats) * gate_big
    aoe = jnp.dot(wup_ref[...], wf.astype(jnp.bfloat16),
                  preferred_element_type=jnp.float32)

    out_ref[0] = xb + shared + aoe

    p_part = jnp.sum(probs, axis=1, keepdims=True)
    l_part = jnp.sum(sel.astype(jnp.float32), axis=1, keepdims=True)

    @pl.when(step == 0)
    def _init():
        psum_ref[...] = jnp.zeros_like(psum_ref)
        lsum_ref[...] = jnp.zeros_like(lsum_ref)

    psum_ref[...] += p_part
    lsum_ref[...] += l_part

    @pl.when(step == nsteps - 1)
    def _fin():
        n_f = jnp.float32(n_tokens)
        aux_ref[...] = (jnp.float32(E) / (n_f * n_f)
                        * jnp.sum(psum_ref[...] * lsum_ref[...], keepdims=True))


def kernel(x, conv1_w, conv1_b, conv2_w, conv2_b, w_down, router_w, w_up):
    B, C, H, W = x.shape
    E, d_low, _ = w_up.shape
    hid = conv1_w.shape[0]
    HW = H * W
    N = B * HW

    x3 = x.reshape(B, C, HW)
    w1b = conv1_w.astype(jnp.bfloat16)
    w2b = conv2_w.astype(jnp.bfloat16)
    eye = jnp.eye(E, dtype=x.dtype)
    rmat = jnp.kron(eye, router_w)
    selm = jnp.kron(eye, jnp.ones((d_low, 1), x.dtype))
    wupt = w_up.reshape(E * d_low, C).T.astype(jnp.bfloat16)

    full = lambda r, c: pl.BlockSpec((r, c), lambda i: (0, 0))
    out3, aux, _, _ = pl.pallas_call(
        functools.partial(_body, B, N),
        grid=(B,),
        in_specs=[
            pl.BlockSpec((1, C, HW), lambda i: (i, 0, 0)),
            full(hid, C), full(hid, 1), full(C, hid), full(C, 1),
            full(E * d_low, C), full(E, E * d_low), full(E * d_low, E),
            full(C, E * d_low),
        ],
        out_specs=[
            pl.BlockSpec((1, C, HW), lambda i: (i, 0, 0)),
            full(1, 1), full(E, 1), full(E, 1),
        ],
        out_shape=[
            jax.ShapeDtypeStruct((B, C, HW), jnp.float32),
            jax.ShapeDtypeStruct((1, 1), jnp.float32),
            jax.ShapeDtypeStruct((E, 1), jnp.float32),
            jax.ShapeDtypeStruct((E, 1), jnp.float32),
        ],
    )(x3, w1b, conv1_b[:, None], w2b, conv2_b[:, None], w_down, rmat, selm,
      wupt)

    return (out3.reshape(B, C, H, W), aux[0, 0])
